# baseline (speedup 1.0000x reference)
import jax
import jax.numpy as jnp
from jax.experimental import pallas as pl
from jax.experimental.pallas import tpu as pltpu

NUM_RAAGS = 1000
EMBED_DIM = 128
SEQ_LEN = 512
BATCH = 1024


def _tile_kernel(idx_ref, row_ref, out_ref):
    out_ref[...] = jnp.broadcast_to(row_ref[...], out_ref.shape)


def kernel(raag_embeddings, table):
    idx = raag_embeddings.reshape(BATCH)
    table3 = table.reshape(NUM_RAAGS, 1, EMBED_DIM)

    grid_spec = pltpu.PrefetchScalarGridSpec(
        num_scalar_prefetch=1,
        grid=(BATCH,),
        in_specs=[
            pl.BlockSpec((1, 1, EMBED_DIM), lambda i, idx_ref: (idx_ref[i], 0, 0)),
        ],
        out_specs=pl.BlockSpec((1, SEQ_LEN, EMBED_DIM), lambda i, idx_ref: (i, 0, 0)),
    )

    out = pl.pallas_call(
        _tile_kernel,
        grid_spec=grid_spec,
        out_shape=jax.ShapeDtypeStruct((BATCH, SEQ_LEN, EMBED_DIM), jnp.float32),
    )(idx, table3)
    return out


# speedup vs baseline: 4.7788x; 4.7788x over previous
import jax
import jax.numpy as jnp
from jax.experimental import pallas as pl
from jax.experimental.pallas import tpu as pltpu

NUM_RAAGS = 1000
EMBED_DIM = 128
SEQ_LEN = 512
BATCH = 1024
B_BLK = 8


def _tile_kernel(idx_ref, table_ref, out_ref):
    i = pl.program_id(0)
    for j in range(B_BLK):
        idx = idx_ref[i * B_BLK + j]
        row = table_ref[pl.ds(idx, 1), :]
        out_ref[j, :, :] = jnp.broadcast_to(row, (SEQ_LEN, EMBED_DIM))


def kernel(raag_embeddings, table):
    idx = raag_embeddings.reshape(BATCH)

    grid_spec = pltpu.PrefetchScalarGridSpec(
        num_scalar_prefetch=1,
        grid=(BATCH // B_BLK,),
        in_specs=[
            pl.BlockSpec((NUM_RAAGS, EMBED_DIM), lambda i, idx_ref: (0, 0)),
        ],
        out_specs=pl.BlockSpec(
            (B_BLK, SEQ_LEN, EMBED_DIM), lambda i, idx_ref: (i, 0, 0)
        ),
    )

    out = pl.pallas_call(
        _tile_kernel,
        grid_spec=grid_spec,
        out_shape=jax.ShapeDtypeStruct((BATCH, SEQ_LEN, EMBED_DIM), jnp.float32),
    )(idx, table)
    return out


# speedup vs baseline: 5.5915x; 1.1701x over previous
import jax
import jax.numpy as jnp
from jax.experimental import pallas as pl
from jax.experimental.pallas import tpu as pltpu

NUM_RAAGS = 1000
EMBED_DIM = 128
SEQ_LEN = 512
BATCH = 1024
B_BLK = 16


def _tile_kernel(idx_ref, table_ref, out_ref):
    i = pl.program_id(0)
    for j in range(B_BLK):
        idx = idx_ref[i * B_BLK + j]
        row = table_ref[pl.ds(idx, 1), :]
        out_ref[j, :, :] = jnp.broadcast_to(row, (SEQ_LEN, EMBED_DIM))


def kernel(raag_embeddings, table):
    idx = raag_embeddings.reshape(BATCH)

    grid_spec = pltpu.PrefetchScalarGridSpec(
        num_scalar_prefetch=1,
        grid=(BATCH // B_BLK,),
        in_specs=[
            pl.BlockSpec((NUM_RAAGS, EMBED_DIM), lambda i, idx_ref: (0, 0)),
        ],
        out_specs=pl.BlockSpec(
            (B_BLK, SEQ_LEN, EMBED_DIM), lambda i, idx_ref: (i, 0, 0)
        ),
    )

    out = pl.pallas_call(
        _tile_kernel,
        grid_spec=grid_spec,
        out_shape=jax.ShapeDtypeStruct((BATCH, SEQ_LEN, EMBED_DIM), jnp.float32),
    )(idx, table)
    return out
